# baseline (speedup 1.0000x reference)
import functools

import jax
import jax.numpy as jnp
from jax import lax
from jax.experimental import pallas as pl
from jax.experimental.pallas import tpu as pltpu
from jax.experimental.pallas import tpu_sc as plsc

N = 10000
E = 320000
G = 64
HID = 128
E_TOT = E + N
ROWS = 2592
E_PAD = ROWS * 128
NW = 32
RPT = ROWS // NW
NP = 10240
NSUB = NP // 16
ER = E // 128
TR = ROWS - ER


def _tc1_body(xp, w1, a1s, a1d, ea1, ea2, we1, aeg1, we2, aeg2,
              h_o, as_o, ad_o, ae1_o, ae2_o, par_o):
    h = jnp.dot(xp[...], w1[...], preferred_element_type=jnp.float32)
    h_o[...] = h
    asr = jnp.sum(h * a1s[...], axis=1, keepdims=True)
    adt = jnp.sum(h * a1d[...], axis=1, keepdims=True)
    as_o[...] = asr
    ad_o[...] = adt

    rr = lax.broadcasted_iota(jnp.int32, (TR, 128), 0)
    cc = lax.broadcasted_iota(jnp.int32, (TR, 128), 1)
    flat = E + rr * 128 + cc
    live = flat < E_TOT

    def edge_coefs(ea_ref, we_ref, aeg_ref, ae_o):
        c = jnp.sum(we_ref[...] * aeg_ref[...])
        m = jnp.sum(ea_ref[...]) / E
        body = c * ea_ref[...]
        ae_o[0:ER, :] = body
        ae_o[ER:ROWS, :] = jnp.where(live, c * m, 0.0)
        return jnp.maximum(jnp.maximum(jnp.max(body), c * m), 0.0)

    mx1 = edge_coefs(ea1, we1, aeg1, ae1_o)
    mx2 = edge_coefs(ea2, we2, aeg2, ae2_o)

    m_pre = jnp.max(asr) + jnp.max(adt) + mx1
    m1 = jnp.maximum(m_pre, 0.2 * m_pre)

    pr = lax.broadcasted_iota(jnp.int32, (8, 128), 0)
    pc = lax.broadcasted_iota(jnp.int32, (8, 128), 1)
    par_o[...] = jnp.where((pr == 0) & (pc == 0), m1,
                           jnp.where((pr == 0) & (pc == 1), mx2, 0.0))


def _tc1(xp, w1, a1s, a1d, ea1, ea2, we1, aeg1, we2, aeg2):
    return pl.pallas_call(
        _tc1_body,
        out_shape=[
            jax.ShapeDtypeStruct((N, HID), jnp.float32),
            jax.ShapeDtypeStruct((N, 1), jnp.float32),
            jax.ShapeDtypeStruct((N, 1), jnp.float32),
            jax.ShapeDtypeStruct((ROWS, 128), jnp.float32),
            jax.ShapeDtypeStruct((ROWS, 128), jnp.float32),
            jax.ShapeDtypeStruct((8, 128), jnp.float32),
        ],
    )(xp, w1, a1s, a1d, ea1, ea2, we1, aeg1, we2, aeg2)


def _sc_body(h_hbm, asrc_hbm, adst_hbm, src_hbm, dst_hbm, ae_hbm, par_hbm,
             outp_hbm, denp_hbm,
             src_v, dst_v, aeb_v, par_v, asg_v, adg_v, exb_v, zrow_v, hbuf_v,
             out_acc, den_acc,
             asem, bsem, esem, dsem_a, dsem_b, gsem_a, gsem_b,
             ssem_a, ssem_b):
    ci = lax.axis_index("c")
    si = lax.axis_index("s")
    wid = ci * 16 + si
    base = wid * RPT

    pltpu.sync_copy(src_hbm.at[wid], src_v)
    pltpu.sync_copy(dst_hbm.at[wid], dst_v)
    pltpu.sync_copy(par_hbm.at[0], par_v)

    zero16 = jnp.zeros((16,), jnp.float32)

    def zrow(j, c):
        for k in range(8):
            hbuf_v[0, j, pl.ds(k * 16, 16)] = zero16
        return c

    lax.fori_loop(0, 64, zrow, 0)
    for k in range(8):
        zrow_v[pl.ds(k * 16, 16)] = zero16
    nb = si * NSUB
    for t in range(10):
        pltpu.sync_copy(hbuf_v.at[0], out_acc.at[pl.ds(nb + t * 64, 64)])
    for t in range(5):
        pltpu.sync_copy(zrow_v, den_acc.at[pl.ds(nb + t * 128, 128)])
    plsc.subcore_barrier()

    zi = jnp.zeros((16,), jnp.int32)
    m_spl = plsc.load_gather(par_v, [zi])
    lane = lax.iota(jnp.int32, 16)

    def attn_gather(r):
        pltpu.async_copy(asrc_hbm.at[src_v.at[r]], asg_v, asem)
        pltpu.async_copy(adst_hbm.at[dst_v.at[r]], adg_v, bsem)
        pltpu.async_copy(ae_hbm.at[wid, r], aeb_v, esem)

    def attn_wait():
        pltpu.make_async_copy(asrc_hbm.at[src_v.at[0]], asg_v, asem).wait()
        pltpu.make_async_copy(adst_hbm.at[dst_v.at[0]], adg_v, bsem).wait()
        pltpu.make_async_copy(ae_hbm.at[wid, 0], aeb_v, esem).wait()

    def hgather(r, b, sem):
        pltpu.async_copy(h_hbm.at[src_v.at[r, pl.ds(b * 64, 64)]],
                         hbuf_v.at[b], sem)

    def hwait(b, sem):
        pltpu.make_async_copy(h_hbm.at[src_v.at[0, pl.ds(b * 64, 64)]],
                              hbuf_v.at[b], sem).wait()

    def den_wait(r, p, dsem):
        pltpu.make_async_copy(exb_v.at[pl.ds(p * 128, 128)],
                              den_acc.at[dst_v.at[r]], dsem).wait()

    def compute_ex(r, p):
        for k in range(8):
            pre = (asg_v[pl.ds(k * 16, 16)] + adg_v[pl.ds(k * 16, 16)]
                   + aeb_v[pl.ds(k * 16, 16)])
            al = jnp.maximum(pre, 0.2 * pre)
            ex = jnp.exp(al - m_spl)
            gidx = (base + r) * 128 + k * 16 + lane
            exb_v[pl.ds(p * 128 + k * 16, 16)] = jnp.where(
                gidx < E_TOT, ex, 0.0)

    def scale(b, p, off):
        for j in range(64):
            s = plsc.load_gather(exb_v, [zi + (p * 128 + off + j)])
            for k in range(8):
                hbuf_v[b, j, pl.ds(k * 16, 16)] = (
                    hbuf_v[b, j, pl.ds(k * 16, 16)] * s)

    def out_scatter(r, b, off, sem):
        return pltpu.async_copy(hbuf_v.at[b],
                                out_acc.at[dst_v.at[r, pl.ds(off, 64)]],
                                sem, add=True)

    def row_stage(r, p, dsem, last):
        attn_wait()
        compute_ex(r, p)
        pltpu.async_copy(exb_v.at[pl.ds(p * 128, 128)],
                         den_acc.at[dst_v.at[r]], dsem, add=True)
        if not last:
            attn_gather(r + 1)
        hwait(0, gsem_a)
        scale(0, p, 0)
        s0 = out_scatter(r, 0, 0, ssem_a)
        hwait(1, gsem_b)
        scale(1, p, 64)
        s1 = out_scatter(r, 1, 64, ssem_b)
        s0.wait()
        if not last:
            hgather(r + 1, 0, gsem_a)
        s1.wait()
        if not last:
            hgather(r + 1, 1, gsem_b)

    attn_gather(0)
    hgather(0, 0, gsem_a)
    hgather(0, 1, gsem_b)
    pltpu.async_copy(zrow_v, den_acc.at[dst_v.at[0]], dsem_a, add=True)
    pltpu.async_copy(zrow_v, den_acc.at[dst_v.at[0]], dsem_b, add=True)

    def pair(pp, c):
        r0 = 2 * pp
        den_wait(r0, 0, dsem_a)
        row_stage(r0, 0, dsem_a, False)
        den_wait(r0 + 1, 1, dsem_b)
        row_stage(r0 + 1, 1, dsem_b, False)
        return c

    lax.fori_loop(0, (RPT - 1) // 2, pair, 0)
    den_wait(RPT - 1, 0, dsem_a)
    row_stage(RPT - 1, 0, dsem_a, True)
    den_wait(RPT - 1, 0, dsem_a)
    den_wait(RPT - 2, 1, dsem_b)

    plsc.subcore_barrier()
    pltpu.sync_copy(out_acc.at[pl.ds(nb, NSUB)],
                    outp_hbm.at[ci, pl.ds(nb, NSUB)])
    pltpu.sync_copy(den_acc.at[pl.ds(nb, NSUB)],
                    denp_hbm.at[ci, pl.ds(nb, NSUB)])


def _sc_edge(h, asrc, adst, src, dst, ae, par):
    mesh = plsc.VectorSubcoreMesh(core_axis_name="c", subcore_axis_name="s",
                                  num_cores=2, num_subcores=16)
    f = pl.kernel(
        _sc_body,
        out_type=[
            jax.ShapeDtypeStruct((2, NP, HID), jnp.float32),
            jax.ShapeDtypeStruct((2, NP), jnp.float32),
        ],
        mesh=mesh,
        compiler_params=pltpu.CompilerParams(needs_layout_passes=False),
        scratch_types=[
            pltpu.VMEM((RPT, 128), jnp.int32),
            pltpu.VMEM((RPT, 128), jnp.int32),
            pltpu.VMEM((128,), jnp.float32),
            pltpu.VMEM((128,), jnp.float32),
            pltpu.VMEM((128,), jnp.float32),
            pltpu.VMEM((128,), jnp.float32),
            pltpu.VMEM((256,), jnp.float32),
            pltpu.VMEM((128,), jnp.float32),
            pltpu.VMEM((2, 64, 128), jnp.float32),
            pltpu.VMEM_SHARED((NP, HID), jnp.float32),
            pltpu.VMEM_SHARED((NP,), jnp.float32),
            pltpu.SemaphoreType.DMA,
            pltpu.SemaphoreType.DMA,
            pltpu.SemaphoreType.DMA,
            pltpu.SemaphoreType.DMA,
            pltpu.SemaphoreType.DMA,
            pltpu.SemaphoreType.DMA,
            pltpu.SemaphoreType.DMA,
            pltpu.SemaphoreType.DMA,
            pltpu.SemaphoreType.DMA,
        ],
    )
    return f(h, asrc, adst, src, dst, ae, par)


def _tc2_body(outp, denp, b1, w2, a2s, a2d, par1,
              h_o, as_o, ad_o, par_o):
    den = denp[0] + denp[1]
    num = outp[0, 0:N, :] + outp[1, 0:N, :]
    out1 = num / (den[0:N, :] + 1e-16)
    r1 = jnp.maximum(out1 + b1[...], 0.0)
    h2 = jnp.dot(r1, w2[...], preferred_element_type=jnp.float32)
    h_o[...] = h2
    asr = jnp.sum(h2 * a2s[...], axis=1, keepdims=True)
    adt = jnp.sum(h2 * a2d[...], axis=1, keepdims=True)
    as_o[...] = asr
    ad_o[...] = adt
    m_pre = jnp.max(asr) + jnp.max(adt) + par1[0, 1]
    m2 = jnp.maximum(m_pre, 0.2 * m_pre)
    pr = lax.broadcasted_iota(jnp.int32, (8, 128), 0)
    pc = lax.broadcasted_iota(jnp.int32, (8, 128), 1)
    par_o[...] = jnp.where((pr == 0) & (pc == 0), m2, 0.0)


def _tc2(outp, denp, b1, w2, a2s, a2d, par1):
    return pl.pallas_call(
        _tc2_body,
        out_shape=[
            jax.ShapeDtypeStruct((N, HID), jnp.float32),
            jax.ShapeDtypeStruct((N, 1), jnp.float32),
            jax.ShapeDtypeStruct((N, 1), jnp.float32),
            jax.ShapeDtypeStruct((8, 128), jnp.float32),
        ],
    )(outp, denp, b1, w2, a2s, a2d, par1)


def _tc3_body(outp, denp, b2, batch, wc, bc, o_ref):
    den = denp[0] + denp[1]
    num = outp[0, 0:N, :] + outp[1, 0:N, :]
    h2 = num / (den[0:N, :] + 1e-16) + b2[...]
    gid = lax.broadcasted_iota(jnp.int32, (G, N), 0)
    onehot = jnp.where(gid == batch[...], 1.0, 0.0)
    s = jnp.dot(onehot, h2, preferred_element_type=jnp.float32)
    cnt = jnp.sum(onehot, axis=1, keepdims=True)
    pooled = s / jnp.maximum(cnt, 1.0)
    logits = jnp.dot(pooled, wc[...],
                     preferred_element_type=jnp.float32) + bc[...]
    mx = jnp.max(logits, axis=1, keepdims=True)
    lse = mx + jnp.log(jnp.sum(jnp.exp(logits - mx), axis=1, keepdims=True))
    o_ref[...] = logits - lse


def _tc3(outp, denp, b2, batch, wc, bc):
    return pl.pallas_call(
        _tc3_body,
        out_shape=jax.ShapeDtypeStruct((G, 2), jnp.float32),
    )(outp, denp, b2, batch, wc, bc)


def kernel(x, edge_index_spatial, edge_weight_spatial, edge_index_temporal,
           edge_weight_temporal, batch, W1, a_src1, a_dst1, a_edge1, We1, b1,
           W2, a_src2, a_dst2, a_edge2, We2, b2, Wc, bc):
    ei1 = edge_index_spatial.astype(jnp.int32)
    ei2 = edge_index_temporal.astype(jnp.int32)
    loop = jnp.arange(N, dtype=jnp.int32)
    padz = jnp.zeros((E_PAD - E_TOT,), jnp.int32)

    def edges(ei):
        s = jnp.concatenate([ei[0], loop, padz]).reshape(NW, RPT, 128)
        d = jnp.concatenate([ei[1], loop, padz]).reshape(NW, RPT, 128)
        return s, d

    src1, dst1 = edges(ei1)
    src2, dst2 = edges(ei2)
    ea1 = edge_weight_spatial.reshape(ER, 128)
    ea2 = edge_weight_temporal.reshape(ER, 128)

    xp = jnp.pad(x, ((0, 0), (0, 7)))
    w1p = jnp.pad(W1, ((0, 7), (0, 0)))

    h1, as1, ad1, ae1f, ae2f, par1 = _tc1(
        xp, w1p, a_src1.reshape(1, HID), a_dst1.reshape(1, HID),
        ea1, ea2, We1.reshape(1, HID), a_edge1.reshape(1, HID),
        We2.reshape(1, HID), a_edge2.reshape(1, HID))

    outp1, denp1 = _sc_edge(h1, as1.reshape(N), ad1.reshape(N),
                            src1, dst1, ae1f.reshape(NW, RPT, 128), par1)

    h2, as2, ad2, par2 = _tc2(outp1, denp1.reshape(2, NP, 1),
                              b1.reshape(1, HID), W2,
                              a_src2.reshape(1, HID), a_dst2.reshape(1, HID),
                              par1)

    outp2, denp2 = _sc_edge(h2, as2.reshape(N), ad2.reshape(N),
                            src2, dst2, ae2f.reshape(NW, RPT, 128), par2)

    return _tc3(outp2, denp2.reshape(2, NP, 1), b2.reshape(1, HID),
                batch.astype(jnp.int32).reshape(1, N), Wc, bc.reshape(1, 2))


# speedup vs baseline: 1.5759x; 1.5759x over previous
import functools

import jax
import jax.numpy as jnp
from jax import lax
from jax.experimental import pallas as pl
from jax.experimental.pallas import tpu as pltpu
from jax.experimental.pallas import tpu_sc as plsc

N = 10000
E = 320000
G = 64
HID = 128
E_TOT = E + N
ROWS = 2592
E_PAD = ROWS * 128
NW = 32
RPT = ROWS // NW
NP = 10240
NSUB = NP // 16
ER = E // 128
TR = ROWS - ER


def _tc1_body(xp, w1, a1s, a1d, ea1, ea2, we1, aeg1, we2, aeg2,
              h_o, as_o, ad_o, ae1_o, ae2_o, par_o):
    h = jnp.dot(xp[...], w1[...], preferred_element_type=jnp.float32)
    h_o[...] = h
    asr = jnp.sum(h * a1s[...], axis=1, keepdims=True)
    adt = jnp.sum(h * a1d[...], axis=1, keepdims=True)
    as_o[...] = asr
    ad_o[...] = adt

    rr = lax.broadcasted_iota(jnp.int32, (TR, 128), 0)
    cc = lax.broadcasted_iota(jnp.int32, (TR, 128), 1)
    flat = E + rr * 128 + cc
    live = flat < E_TOT

    def edge_coefs(ea_ref, we_ref, aeg_ref, ae_o):
        c = jnp.sum(we_ref[...] * aeg_ref[...])
        m = jnp.sum(ea_ref[...]) / E
        body = c * ea_ref[...]
        ae_o[0:ER, :] = body
        ae_o[ER:ROWS, :] = jnp.where(live, c * m, 0.0)
        return jnp.maximum(jnp.maximum(jnp.max(body), c * m), 0.0)

    mx1 = edge_coefs(ea1, we1, aeg1, ae1_o)
    mx2 = edge_coefs(ea2, we2, aeg2, ae2_o)

    m_pre = jnp.max(asr) + jnp.max(adt) + mx1
    m1 = jnp.maximum(m_pre, 0.2 * m_pre)

    pr = lax.broadcasted_iota(jnp.int32, (8, 128), 0)
    pc = lax.broadcasted_iota(jnp.int32, (8, 128), 1)
    par_o[...] = jnp.where((pr == 0) & (pc == 0), m1,
                           jnp.where((pr == 0) & (pc == 1), mx2, 0.0))


def _tc1(xp, w1, a1s, a1d, ea1, ea2, we1, aeg1, we2, aeg2):
    return pl.pallas_call(
        _tc1_body,
        out_shape=[
            jax.ShapeDtypeStruct((N, HID), jnp.float32),
            jax.ShapeDtypeStruct((N, 1), jnp.float32),
            jax.ShapeDtypeStruct((N, 1), jnp.float32),
            jax.ShapeDtypeStruct((ROWS, 128), jnp.float32),
            jax.ShapeDtypeStruct((ROWS, 128), jnp.float32),
            jax.ShapeDtypeStruct((8, 128), jnp.float32),
        ],
    )(xp, w1, a1s, a1d, ea1, ea2, we1, aeg1, we2, aeg2)


def _sc_body(h_hbm, asrc_hbm, adst_hbm, src_hbm, dst_hbm, ae_hbm, par_hbm,
             outp_hbm, denp_hbm,
             src_v, dst_v, aeb_v, par_v, asg_v, adg_v, exb_v, zrow_v, hbuf_v,
             out_acc, den_acc,
             asem, bsem, esem, dsem_a, dsem_b, gsem_a, gsem_b,
             ssem_a, ssem_b):
    ci = lax.axis_index("c")
    si = lax.axis_index("s")
    wid = ci * 16 + si
    base = wid * RPT

    pltpu.sync_copy(src_hbm.at[wid], src_v)
    pltpu.sync_copy(dst_hbm.at[wid], dst_v)
    pltpu.sync_copy(par_hbm.at[0], par_v)

    zero16 = jnp.zeros((16,), jnp.float32)

    def zrow(j, c):
        for k in range(8):
            hbuf_v[0, j, pl.ds(k * 16, 16)] = zero16
        return c

    lax.fori_loop(0, 64, zrow, 0)
    for k in range(8):
        zrow_v[pl.ds(k * 16, 16)] = zero16
    nb = si * NSUB
    for t in range(10):
        pltpu.sync_copy(hbuf_v.at[0], out_acc.at[pl.ds(nb + t * 64, 64)])
    for t in range(5):
        pltpu.sync_copy(zrow_v, den_acc.at[pl.ds(nb + t * 128, 128)])
    plsc.subcore_barrier()

    zi = jnp.zeros((16,), jnp.int32)
    m_spl = plsc.load_gather(par_v, [zi])
    lane = lax.iota(jnp.int32, 16)

    def attn_gather(r):
        pltpu.async_copy(asrc_hbm.at[src_v.at[r]], asg_v, asem)
        pltpu.async_copy(adst_hbm.at[dst_v.at[r]], adg_v, bsem)
        pltpu.async_copy(ae_hbm.at[wid, r], aeb_v, esem)

    def attn_wait():
        pltpu.make_async_copy(asrc_hbm.at[src_v.at[0]], asg_v, asem).wait()
        pltpu.make_async_copy(adst_hbm.at[dst_v.at[0]], adg_v, bsem).wait()
        pltpu.make_async_copy(ae_hbm.at[wid, 0], aeb_v, esem).wait()

    def hgather(r, b, sem):
        pltpu.async_copy(h_hbm.at[src_v.at[r, pl.ds(b * 64, 64)]],
                         hbuf_v.at[b], sem)

    def hwait(b, sem):
        pltpu.make_async_copy(h_hbm.at[src_v.at[0, pl.ds(b * 64, 64)]],
                              hbuf_v.at[b], sem).wait()

    def den_wait(r, p, dsem):
        pltpu.make_async_copy(exb_v.at[pl.ds(p * 128, 128)],
                              den_acc.at[dst_v.at[r]], dsem).wait()

    def compute_ex(r, p):
        for k in range(8):
            pre = (asg_v[pl.ds(k * 16, 16)] + adg_v[pl.ds(k * 16, 16)]
                   + aeb_v[pl.ds(k * 16, 16)])
            al = jnp.maximum(pre, 0.2 * pre)
            ex = jnp.exp(al - m_spl)
            gidx = (base + r) * 128 + k * 16 + lane
            exb_v[pl.ds(p * 128 + k * 16, 16)] = jnp.where(
                gidx < E_TOT, ex, 0.0)

    def scale(b, p, off):
        def body(j, c):
            j4 = j * 4
            for u in range(4):
                s = plsc.load_gather(exb_v, [zi + (p * 128 + off + j4 + u)])
                for k in range(8):
                    hbuf_v[b, j4 + u, pl.ds(k * 16, 16)] = (
                        hbuf_v[b, j4 + u, pl.ds(k * 16, 16)] * s)
            return c

        lax.fori_loop(0, 16, body, 0)

    def out_scatter(r, b, off, sem):
        return pltpu.async_copy(hbuf_v.at[b],
                                out_acc.at[dst_v.at[r, pl.ds(off, 64)]],
                                sem, add=True)

    def row_stage(r, p, dsem, last):
        attn_wait()
        compute_ex(r, p)
        pltpu.async_copy(exb_v.at[pl.ds(p * 128, 128)],
                         den_acc.at[dst_v.at[r]], dsem, add=True)
        if not last:
            attn_gather(r + 1)
        hwait(0, gsem_a)
        scale(0, p, 0)
        s0 = out_scatter(r, 0, 0, ssem_a)
        hwait(1, gsem_b)
        scale(1, p, 64)
        s1 = out_scatter(r, 1, 64, ssem_b)
        s0.wait()
        if not last:
            hgather(r + 1, 0, gsem_a)
        s1.wait()
        if not last:
            hgather(r + 1, 1, gsem_b)

    attn_gather(0)
    hgather(0, 0, gsem_a)
    hgather(0, 1, gsem_b)
    pltpu.async_copy(zrow_v, den_acc.at[dst_v.at[0]], dsem_a, add=True)
    pltpu.async_copy(zrow_v, den_acc.at[dst_v.at[0]], dsem_b, add=True)

    def pair(pp, c):
        r0 = 2 * pp
        den_wait(r0, 0, dsem_a)
        row_stage(r0, 0, dsem_a, False)
        den_wait(r0 + 1, 1, dsem_b)
        row_stage(r0 + 1, 1, dsem_b, False)
        return c

    lax.fori_loop(0, (RPT - 1) // 2, pair, 0)
    den_wait(RPT - 1, 0, dsem_a)
    row_stage(RPT - 1, 0, dsem_a, True)
    den_wait(RPT - 1, 0, dsem_a)
    den_wait(RPT - 2, 1, dsem_b)

    plsc.subcore_barrier()
    pltpu.sync_copy(out_acc.at[pl.ds(nb, NSUB)],
                    outp_hbm.at[ci, pl.ds(nb, NSUB)])
    pltpu.sync_copy(den_acc.at[pl.ds(nb, NSUB)],
                    denp_hbm.at[ci, pl.ds(nb, NSUB)])


def _sc_edge(h, asrc, adst, src, dst, ae, par):
    mesh = plsc.VectorSubcoreMesh(core_axis_name="c", subcore_axis_name="s",
                                  num_cores=2, num_subcores=16)
    f = pl.kernel(
        _sc_body,
        out_type=[
            jax.ShapeDtypeStruct((2, NP, HID), jnp.float32),
            jax.ShapeDtypeStruct((2, NP), jnp.float32),
        ],
        mesh=mesh,
        compiler_params=pltpu.CompilerParams(needs_layout_passes=False),
        scratch_types=[
            pltpu.VMEM((RPT, 128), jnp.int32),
            pltpu.VMEM((RPT, 128), jnp.int32),
            pltpu.VMEM((128,), jnp.float32),
            pltpu.VMEM((128,), jnp.float32),
            pltpu.VMEM((128,), jnp.float32),
            pltpu.VMEM((128,), jnp.float32),
            pltpu.VMEM((256,), jnp.float32),
            pltpu.VMEM((128,), jnp.float32),
            pltpu.VMEM((2, 64, 128), jnp.float32),
            pltpu.VMEM_SHARED((NP, HID), jnp.float32),
            pltpu.VMEM_SHARED((NP,), jnp.float32),
            pltpu.SemaphoreType.DMA,
            pltpu.SemaphoreType.DMA,
            pltpu.SemaphoreType.DMA,
            pltpu.SemaphoreType.DMA,
            pltpu.SemaphoreType.DMA,
            pltpu.SemaphoreType.DMA,
            pltpu.SemaphoreType.DMA,
            pltpu.SemaphoreType.DMA,
            pltpu.SemaphoreType.DMA,
        ],
    )
    return f(h, asrc, adst, src, dst, ae, par)


def _tc2_body(outp, denp, b1, w2, a2s, a2d, par1,
              h_o, as_o, ad_o, par_o):
    den = denp[0] + denp[1]
    num = outp[0, 0:N, :] + outp[1, 0:N, :]
    out1 = num / (den[0:N, :] + 1e-16)
    r1 = jnp.maximum(out1 + b1[...], 0.0)
    h2 = jnp.dot(r1, w2[...], preferred_element_type=jnp.float32)
    h_o[...] = h2
    asr = jnp.sum(h2 * a2s[...], axis=1, keepdims=True)
    adt = jnp.sum(h2 * a2d[...], axis=1, keepdims=True)
    as_o[...] = asr
    ad_o[...] = adt
    m_pre = jnp.max(asr) + jnp.max(adt) + par1[0, 1]
    m2 = jnp.maximum(m_pre, 0.2 * m_pre)
    pr = lax.broadcasted_iota(jnp.int32, (8, 128), 0)
    pc = lax.broadcasted_iota(jnp.int32, (8, 128), 1)
    par_o[...] = jnp.where((pr == 0) & (pc == 0), m2, 0.0)


def _tc2(outp, denp, b1, w2, a2s, a2d, par1):
    return pl.pallas_call(
        _tc2_body,
        out_shape=[
            jax.ShapeDtypeStruct((N, HID), jnp.float32),
            jax.ShapeDtypeStruct((N, 1), jnp.float32),
            jax.ShapeDtypeStruct((N, 1), jnp.float32),
            jax.ShapeDtypeStruct((8, 128), jnp.float32),
        ],
    )(outp, denp, b1, w2, a2s, a2d, par1)


def _tc3_body(outp, denp, b2, batch, wc, bc, o_ref):
    den = denp[0] + denp[1]
    num = outp[0, 0:N, :] + outp[1, 0:N, :]
    h2 = num / (den[0:N, :] + 1e-16) + b2[...]
    gid = lax.broadcasted_iota(jnp.int32, (G, N), 0)
    onehot = jnp.where(gid == batch[...], 1.0, 0.0)
    s = jnp.dot(onehot, h2, preferred_element_type=jnp.float32)
    cnt = jnp.sum(onehot, axis=1, keepdims=True)
    pooled = s / jnp.maximum(cnt, 1.0)
    logits = jnp.dot(pooled, wc[...],
                     preferred_element_type=jnp.float32) + bc[...]
    mx = jnp.max(logits, axis=1, keepdims=True)
    lse = mx + jnp.log(jnp.sum(jnp.exp(logits - mx), axis=1, keepdims=True))
    o_ref[...] = logits - lse


def _tc3(outp, denp, b2, batch, wc, bc):
    return pl.pallas_call(
        _tc3_body,
        out_shape=jax.ShapeDtypeStruct((G, 2), jnp.float32),
    )(outp, denp, b2, batch, wc, bc)


def kernel(x, edge_index_spatial, edge_weight_spatial, edge_index_temporal,
           edge_weight_temporal, batch, W1, a_src1, a_dst1, a_edge1, We1, b1,
           W2, a_src2, a_dst2, a_edge2, We2, b2, Wc, bc):
    ei1 = edge_index_spatial.astype(jnp.int32)
    ei2 = edge_index_temporal.astype(jnp.int32)
    loop = jnp.arange(N, dtype=jnp.int32)
    padz = jnp.zeros((E_PAD - E_TOT,), jnp.int32)

    def edges(ei):
        s = jnp.concatenate([ei[0], loop, padz]).reshape(NW, RPT, 128)
        d = jnp.concatenate([ei[1], loop, padz]).reshape(NW, RPT, 128)
        return s, d

    src1, dst1 = edges(ei1)
    src2, dst2 = edges(ei2)
    ea1 = edge_weight_spatial.reshape(ER, 128)
    ea2 = edge_weight_temporal.reshape(ER, 128)

    xp = jnp.pad(x, ((0, 0), (0, 7)))
    w1p = jnp.pad(W1, ((0, 7), (0, 0)))

    h1, as1, ad1, ae1f, ae2f, par1 = _tc1(
        xp, w1p, a_src1.reshape(1, HID), a_dst1.reshape(1, HID),
        ea1, ea2, We1.reshape(1, HID), a_edge1.reshape(1, HID),
        We2.reshape(1, HID), a_edge2.reshape(1, HID))

    outp1, denp1 = _sc_edge(h1, as1.reshape(N), ad1.reshape(N),
                            src1, dst1, ae1f.reshape(NW, RPT, 128), par1)

    h2, as2, ad2, par2 = _tc2(outp1, denp1.reshape(2, NP, 1),
                              b1.reshape(1, HID), W2,
                              a_src2.reshape(1, HID), a_dst2.reshape(1, HID),
                              par1)

    outp2, denp2 = _sc_edge(h2, as2.reshape(N), ad2.reshape(N),
                            src2, dst2, ae2f.reshape(NW, RPT, 128), par2)

    return _tc3(outp2, denp2.reshape(2, NP, 1), b2.reshape(1, HID),
                batch.astype(jnp.int32).reshape(1, N), Wc, bc.reshape(1, 2))
